# baseline (speedup 1.0000x reference)
import jax
import jax.numpy as jnp
from jax import lax
from jax.experimental import pallas as pl
from jax.experimental.pallas import tpu as pltpu
from jax.experimental.pallas import tpu_sc as plsc

MODEL_DIM = 64
NUM_WORKERS = 32
CHUNK = 256
NBUF = 4


def kernel(x, table):
    batch, hist = x.shape
    n = batch * hist
    idx = x.reshape(n).astype(jnp.int32)

    per_w = n // NUM_WORKERS
    nchunks = per_w // CHUNK

    mesh = plsc.VectorSubcoreMesh(core_axis_name="core",
                                  subcore_axis_name="subcore")

    @pl.kernel(out_type=jax.ShapeDtypeStruct((n, MODEL_DIM), table.dtype),
               mesh=mesh,
               scratch_types=[
                   pltpu.VMEM((per_w,), jnp.int32),
                   pltpu.VMEM((NBUF, CHUNK, MODEL_DIM), jnp.float32),
                   pltpu.SemaphoreType.DMA((NBUF,)),
                   pltpu.SemaphoreType.DMA((NBUF,)),
               ],
               compiler_params=pltpu.CompilerParams(use_tc_tiling_on_sc=False))
    def gather_kernel(table_hbm, idx_hbm, out_hbm, idx_v, rows_v, gsem, ssem):
        wid = lax.axis_index("subcore") * 2 + lax.axis_index("core")
        base = wid * per_w
        pltpu.sync_copy(idx_hbm.at[pl.ds(base, per_w)], idx_v)

        def gather(g, b):
            pltpu.make_async_copy(
                table_hbm.at[idx_v.at[pl.ds(g * CHUNK, CHUNK)]],
                rows_v.at[b], gsem.at[b]).start()

        def store(g, b):
            pltpu.make_async_copy(
                rows_v.at[b],
                out_hbm.at[pl.ds(base + g * CHUNK, CHUNK)],
                ssem.at[b]).start()

        def wait_gather(g, b):
            pltpu.make_async_copy(
                table_hbm.at[idx_v.at[pl.ds(g * CHUNK, CHUNK)]],
                rows_v.at[b], gsem.at[b]).wait()

        def wait_store(g, b):
            pltpu.make_async_copy(
                rows_v.at[b],
                out_hbm.at[pl.ds(base + g * CHUNK, CHUNK)],
                ssem.at[b]).wait()

        for b in range(NBUF):
            gather(b, b)

        @pl.loop(0, nchunks - NBUF, step=NBUF)
        def _(g0):
            for b in range(NBUF):
                g = g0 + b
                wait_gather(g, b)
                store(g, b)
            for b in range(NBUF):
                g = g0 + b
                wait_store(g, b)
                gather(g + NBUF, b)

        for b in range(NBUF):
            g = nchunks - NBUF + b
            wait_gather(g, b)
            store(g, b)
        for b in range(NBUF):
            g = nchunks - NBUF + b
            wait_store(g, b)

    out = gather_kernel(table, idx)
    return out.reshape(batch, hist, MODEL_DIM)
